# Initial kernel scaffold; baseline (speedup 1.0000x reference)
import jax
import jax.numpy as jnp
from jax.experimental import pallas as pl


def kernel(nn_vecs, species, inda, indb, inde, mask, nats, emb, W_up, W_m1, W_m2, W_m3, W_m4, W_dn, W_sc, W_post, W_skip, W_ro0, W_mlp, W_ro1):
    raise NotImplementedError("write your pallas kernel here")


# speedup vs baseline: 4.7095x; 4.7095x over previous
import math

import jax
import jax.numpy as jnp
import numpy as np
from jax.experimental import pallas as pl
from jax.experimental.pallas import tpu as pltpu
from jax.experimental.pallas import tpu_sc as plsc

N = 10000
E = 160000
F = 32
S = 8
G = 64
RMAX = 5.0
EPS = 0.1
_SQS = math.sqrt(float(S))
_OFFS = np.arange(0.0, -8.0, -1.0, dtype=np.float32).reshape(8, 1)

NC = 2
NS = 16
NW = NC * NS
CH = 128
N_PAD = 10240
E_PAD = 163840

BN = 1000
BE = 2048
BEB = 512

_SC_PARAMS = pltpu.CompilerParams(use_tc_tiling_on_sc=False)


def _pcall(f, grid, in_specs, out_specs, out_shape):
    return pl.pallas_call(f, grid=grid, in_specs=in_specs,
                          out_specs=out_specs, out_shape=out_shape)


def _sc_gather(table, idx, width):
    rows_per_w = E_PAD // NW
    nch = rows_per_w // CH
    mesh = plsc.VectorSubcoreMesh(core_axis_name="c", subcore_axis_name="s")

    def body(table_hbm, idx_hbm, out_hbm, idx_v, rows_v, sem):
        wid = jax.lax.axis_index("s") * NC + jax.lax.axis_index("c")
        base = wid * rows_per_w

        def step(i, carry):
            off = base + i * CH
            pltpu.sync_copy(idx_hbm.at[pl.ds(off, CH)], idx_v)
            pltpu.async_copy(table_hbm.at[idx_v], rows_v, sem).wait()
            pltpu.sync_copy(rows_v, out_hbm.at[pl.ds(off, CH)])
            return carry

        jax.lax.fori_loop(0, nch, step, 0)

    run = pl.kernel(
        body,
        out_type=jax.ShapeDtypeStruct((E_PAD, width), jnp.float32),
        mesh=mesh,
        compiler_params=_SC_PARAMS,
        scratch_types=[
            pltpu.VMEM((CH,), jnp.int32),
            pltpu.VMEM((CH, width), jnp.float32),
            pltpu.SemaphoreType.DMA,
        ],
    )
    return run(table, idx)


def _sc_scatter(vals, idx, width):
    vrows = vals.shape[0]
    rows_per_w = vrows // NW
    nch = rows_per_w // CH
    rows_per_sub = N_PAD // NS
    zeros = jnp.zeros((N_PAD, width), jnp.float32)
    mesh = plsc.VectorSubcoreMesh(core_axis_name="c", subcore_axis_name="s")

    def body(vals_hbm, idx_hbm, zeros_hbm, out_hbm, idx_v, vals_v, shared):
        cid = jax.lax.axis_index("c")
        sid = jax.lax.axis_index("s")
        wid = sid * NC + cid
        pltpu.sync_copy(zeros_hbm.at[pl.ds(sid * rows_per_sub, rows_per_sub)],
                        shared.at[pl.ds(sid * rows_per_sub, rows_per_sub)])
        plsc.subcore_barrier()
        base = wid * rows_per_w

        def step(i, carry):
            off = base + i * CH
            pltpu.sync_copy(idx_hbm.at[pl.ds(off, CH)], idx_v)
            pltpu.sync_copy(vals_hbm.at[pl.ds(off, CH)], vals_v)
            pltpu.sync_copy(vals_v, shared.at[idx_v], add=True)
            return carry

        jax.lax.fori_loop(0, nch, step, 0)
        plsc.subcore_barrier()
        pltpu.sync_copy(shared.at[pl.ds(sid * rows_per_sub, rows_per_sub)],
                        out_hbm.at[cid, pl.ds(sid * rows_per_sub, rows_per_sub)])

    run = pl.kernel(
        body,
        out_type=jax.ShapeDtypeStruct((NC, N_PAD, width), jnp.float32),
        mesh=mesh,
        compiler_params=_SC_PARAMS,
        scratch_types=[
            pltpu.VMEM((CH,), jnp.int32),
            pltpu.VMEM((CH, width), jnp.float32),
            pltpu.VMEM_SHARED((N_PAD, width), jnp.float32),
        ],
    )
    return run(vals, idx, zeros)


def _dot(a, b):
    return jnp.dot(a, b, preferred_element_type=jnp.float32)


def _split4(a):
    return a[:, 0:32], a[:, 32:64], a[:, 64:96], a[:, 96:128]


def _edge_math(vecs, xs, wm1, wm2, wm3, wm4r):
    vx, vy, vz = vecs[:, 0:1], vecs[:, 1:2], vecs[:, 2:3]
    r = jnp.sqrt(vx * vx + vy * vy + vz * vz)
    xr = jnp.maximum(r, 1e-9)
    ux, uy, uz = vx / xr, vy / xr, vz / xr
    ns = jax.lax.broadcasted_iota(jnp.int32, (1, 8), 1).astype(
        jnp.float32) + 1.0
    b = np.float32(np.sqrt(2.0 / RMAX)) * jnp.sin(ns * (np.pi / RMAX) * xr) / xr
    t = r * (1.0 / RMAX)
    xp = t * t * t * t * t
    env = 1.0 - 21.0 * xp + 35.0 * xp * t - 15.0 * xp * t * t
    cut = jnp.where(r < RMAX, env, 0.0)
    rad = b * cut
    h = jax.nn.silu(_dot(rad, wm1))
    h = jax.nn.silu(_dot(h, wm2))
    h = jax.nn.silu(_dot(h, wm3))
    mixr = _dot(h, wm4r)
    mix0, mix1, mix2, mix3, mix4 = (mixr[:, 32 * j:32 * j + 32]
                                    for j in range(5))
    x0, x1, x2, x3 = _split4(xs)
    dotv = x1 * ux + x2 * uy + x3 * uz
    m0 = mix0 * x0 + mix1 * dotv
    c1 = x2 * uz - x3 * uy
    c2 = x3 * ux - x1 * uz
    c3 = x1 * uy - x2 * ux
    mv1 = mix2 * x0 * ux + mix3 * x1 + mix4 * c1
    mv2 = mix2 * x0 * uy + mix3 * x2 + mix4 * c2
    mv3 = mix2 * x0 * uz + mix3 * x3 + mix4 * c3
    return jnp.concatenate([m0, mv1, mv2, mv3], axis=1) * EPS


def _node0_math(agg, oh, wdn0, wdn1, wskip0, wsc0r, wpost0, wpost1, wro0,
                wup10, wup11):
    a0, a1, a2, a3 = _split4(agg)
    y0 = _dot(a0, wdn0)
    yv = [_dot(a, wdn1) for a in (a1, a2, a3)]

    def gcontract(tq):
        acc = jnp.zeros_like(tq)
        for s in range(S):
            acc = acc + oh[:, s:s + 1] * _dot(tq, wskip0[s])
        return acc * (1.0 / _SQS)

    y0 = gcontract(y0)
    yv = [gcontract(y) for y in yv]
    ws = _dot(oh, wsc0r)
    ws0, ws1 = ws[:, 0:32], ws[:, 32:64]
    z0 = ws0 * y0 + ws1 * y0 * y0
    p0 = _dot(z0, wpost0)
    pv = [_dot(ws0 * y, wpost1) for y in yv]
    e0 = _dot(p0, wro0)
    tx1 = jnp.concatenate([_dot(p0, wup10)] + [_dot(p, wup11) for p in pv],
                          axis=1)
    return tx1, e0


def _e1_math(a0, oh, wdn10, wsc1r, wpost10, wmlp, wro1):
    y0 = _dot(a0, wdn10)
    ws = _dot(oh, wsc1r)
    z0 = ws[:, 0:32] * y0 + ws[:, 32:64] * y0 * y0
    p0 = _dot(z0, wpost10)
    h = jax.nn.silu(_dot(p0, wmlp))
    return _dot(h, wro1)


def _onehot(sp_ref, k):
    sp = sp_ref[0, 0, :]
    ids = jax.lax.broadcasted_iota(jnp.int32, (sp.shape[0], k), 1).astype(
        jnp.float32)
    return jnp.where(sp[:, None] == ids, 1.0, 0.0)


def _full(shape):
    return pl.BlockSpec(shape, lambda i: tuple(0 for _ in shape))


def _rows(bs, w):
    return pl.BlockSpec((bs, w), lambda i: (i, 0))


def _sp3(bs):
    return pl.BlockSpec((1, 1, bs), lambda i: (i, 0, 0))


def _k_init(species3, emb, wup00):
    def body(sp_ref, emb_ref, w_ref, tx_ref):
        oh = _onehot(sp_ref, S)
        x00 = _dot(oh, emb_ref[...]) * (1.0 / _SQS)
        tx_ref[...] = _dot(x00, w_ref[...])

    return _pcall(body, (N // BN,),
                  [_sp3(BN), _full((S, F)), _full((F, F))],
                  _rows(BN, 32),
                  jax.ShapeDtypeStruct((N, 32), jnp.float32))(
                      species3, emb, wup00)


def _k_edge_fwd(vecs_p, xs, wm1, wm2, wm3, wm4r, narrow):
    def body(v_ref, xs_ref, w1, w2, w3, w4, m_ref):
        xsv = xs_ref[...]
        if narrow:
            xsv = jnp.concatenate(
                [xsv, jnp.zeros((xsv.shape[0], 96), jnp.float32)], axis=1)
        m_ref[...] = _edge_math(v_ref[...], xsv, w1[...], w2[...], w3[...],
                                w4[...])

    xw = 32 if narrow else 128
    return _pcall(body, (E_PAD // BE,),
                  [_rows(BE, 3), _rows(BE, xw), _full((8, 64)),
                   _full((64, 64)), _full((64, 64)), _full((64, 160))],
                  _rows(BE, 128),
                  jax.ShapeDtypeStruct((E_PAD, 128), jnp.float32))(
                      vecs_p, xs, wm1, wm2, wm3, wm4r)


def _k_edge_bwd(vecs_p, xs, dm, wm1, wm2, wm3, wm4r, narrow, want_dxs):
    def body(v_ref, xs_ref, dm_ref, w1, w2, w3, w4, *outs):
        xsv = xs_ref[...]
        if narrow:
            xsv = jnp.concatenate(
                [xsv, jnp.zeros((xsv.shape[0], 96), jnp.float32)], axis=1)
        dmv = dm_ref[...]
        if dmv.shape[1] == 32:
            dmv = jnp.concatenate(
                [dmv, jnp.zeros((dmv.shape[0], 96), jnp.float32)], axis=1)
        w1v, w2v, w3v, w4v = w1[...], w2[...], w3[...], w4[...]
        fn = lambda v, x: _edge_math(v, x, w1v, w2v, w3v, w4v)
        _, vjpf = jax.vjp(fn, v_ref[...], xsv)
        dv, dxs = vjpf(dmv)
        outs[0][...] = dv
        if want_dxs:
            outs[1][...] = dxs

    xw = 32 if narrow else 128
    dmw = dm.shape[1]
    out_shape = [jax.ShapeDtypeStruct((E_PAD, 3), jnp.float32)]
    out_specs = [_rows(BEB, 3)]
    if want_dxs:
        out_shape.append(jax.ShapeDtypeStruct((E_PAD, 128), jnp.float32))
        out_specs.append(_rows(BEB, 128))
    return _pcall(body, (E_PAD // BEB,),
                  [_rows(BEB, 3), _rows(BEB, xw), _rows(BEB, dmw),
                   _full((8, 64)), _full((64, 64)), _full((64, 64)),
                   _full((64, 160))],
                  out_specs, out_shape)(vecs_p, xs, dm, wm1, wm2, wm3, wm4r)


def _k_node0_fwd(p0t, p1t, species3, w):
    def body(p0_ref, p1_ref, sp_ref, wdn0, wdn1, wskip0, wsc0r, wpost0,
             wpost1, wro0, wup10, wup11, tx_ref, e_ref):
        agg = p0_ref[...] + p1_ref[...]
        oh = _onehot(sp_ref, S)
        tx1, e0 = _node0_math(agg, oh, wdn0[...], wdn1[...], wskip0[...],
                              wsc0r[...], wpost0[...], wpost1[...], wro0[...],
                              wup10[...], wup11[...])
        tx_ref[...] = tx1
        e_ref[...] = e0

    return _pcall(body, (N // BN,),
                  [_rows(BN, 128), _rows(BN, 128), _sp3(BN),
                   _full((F, F)), _full((F, F)), _full((S, F, F)),
                   _full((S, 2 * F)), _full((F, F)), _full((F, F)),
                   _full((F, 1)), _full((F, F)), _full((F, F))],
                  [_rows(BN, 128), _rows(BN, 1)],
                  [jax.ShapeDtypeStruct((N, 128), jnp.float32),
                   jax.ShapeDtypeStruct((N, 1), jnp.float32)])(
                      p0t, p1t, species3, *w)


def _k_node0_bwd(p0t, p1t, species3, dtx0, dtx1, w):
    def body(p0_ref, p1_ref, sp_ref, dt0_ref, dt1_ref, wdn0, wdn1, wskip0,
             wsc0r, wpost0, wpost1, wro0, wup10, wup11, dagg_ref):
        agg = p0_ref[...] + p1_ref[...]
        oh = _onehot(sp_ref, S)
        args = (wdn0[...], wdn1[...], wskip0[...], wsc0r[...], wpost0[...],
                wpost1[...], wro0[...], wup10[...], wup11[...])
        fn = lambda a: _node0_math(a, oh, *args)
        _, vjpf = jax.vjp(fn, agg)
        dtx = dt0_ref[...] + dt1_ref[...]
        (dagg,) = vjpf((dtx, jnp.ones((agg.shape[0], 1), jnp.float32)))
        dagg_ref[...] = dagg

    return _pcall(body, (N // BN,),
                  [_rows(BN, 128), _rows(BN, 128), _sp3(BN),
                   _rows(BN, 128), _rows(BN, 128),
                   _full((F, F)), _full((F, F)), _full((S, F, F)),
                   _full((S, 2 * F)), _full((F, F)), _full((F, F)),
                   _full((F, 1)), _full((F, F)), _full((F, F))],
                  _rows(BN, 128),
                  jax.ShapeDtypeStruct((N, 128), jnp.float32))(
                      p0t, p1t, species3, dtx0, dtx1, *w)


def _k_final_e(a0p0, a0p1, species3, inde3, e0, w):
    def body(p0_ref, p1_ref, sp_ref, ge_ref, e0_ref, wdn10, wsc1r, wpost10,
             wmlp, wro1, offs_ref, eg_ref):
        i = pl.program_id(0)
        a0 = p0_ref[...] + p1_ref[...]
        oh = _onehot(sp_ref, S)
        e1 = _e1_math(a0, oh, wdn10[...], wsc1r[...], wpost10[...],
                      wmlp[...], wro1[...])
        off = _dot(oh, offs_ref[...])
        ei = e0_ref[...] + e1 + off
        ohg = _onehot(ge_ref, G)
        blk = _dot(ei.reshape(1, ei.shape[0]), ohg)

        @pl.when(i == 0)
        def _():
            eg_ref[...] = jnp.zeros_like(eg_ref)

        eg_ref[...] += blk

    return _pcall(body, (N // BN,),
                  [_rows(BN, 32), _rows(BN, 32), _sp3(BN), _sp3(BN),
                   _rows(BN, 1), _full((F, F)), _full((S, 2 * F)),
                   _full((F, F)), _full((F, 16)), _full((16, 1)),
                   _full((S, 1))],
                  pl.BlockSpec((1, G), lambda i: (0, 0)),
                  jax.ShapeDtypeStruct((1, G), jnp.float32))(
                      a0p0, a0p1, species3, inde3, e0, *w,
                      jnp.asarray(_OFFS))


def _k_node1_bwd(a0p0, a0p1, species3, w):
    def body(p0_ref, p1_ref, sp_ref, wdn10, wsc1r, wpost10, wmlp, wro1,
             da_ref):
        a0 = p0_ref[...] + p1_ref[...]
        oh = _onehot(sp_ref, S)
        args = (wdn10[...], wsc1r[...], wpost10[...], wmlp[...], wro1[...])
        fn = lambda a: _e1_math(a, oh, *args)
        _, vjpf = jax.vjp(fn, a0)
        (da,) = vjpf(jnp.ones((a0.shape[0], 1), jnp.float32))
        da_ref[...] = da

    return _pcall(body, (N // BN,),
                  [_rows(BN, 32), _rows(BN, 32), _sp3(BN), _full((F, F)),
                   _full((S, 2 * F)), _full((F, F)), _full((F, 16)),
                   _full((16, 1))],
                  _rows(BN, 32),
                  jax.ShapeDtypeStruct((N, 32), jnp.float32))(
                      a0p0, a0p1, species3, *w)


def _k_fo_vals(dv0, dv1, mask_p):
    def body(d0_ref, d1_ref, m_ref, va_ref, vb_ref):
        ft = (d0_ref[...] + d1_ref[...]) * m_ref[...]
        pad = jnp.zeros((ft.shape[0], 13), jnp.float32)
        va = jnp.concatenate([ft, pad], axis=1)
        va_ref[...] = va
        vb_ref[...] = -va

    return _pcall(body, (E_PAD // BE,),
                  [_rows(BE, 3), _rows(BE, 3), _rows(BE, 1)],
                  [_rows(BE, 16), _rows(BE, 16)],
                  [jax.ShapeDtypeStruct((E_PAD, 16), jnp.float32),
                   jax.ShapeDtypeStruct((E_PAD, 16), jnp.float32)])(
                      dv0, dv1, mask_p)


def kernel(nn_vecs, species, inda, indb, inde, mask, nats, emb, W_up, W_m1,
           W_m2, W_m3, W_m4, W_dn, W_sc, W_post, W_skip, W_ro0, W_mlp,
           W_ro1):
    f32 = jnp.float32
    pe = E_PAD - E

    vecs_p = jnp.concatenate(
        [nn_vecs.astype(f32), jnp.ones((pe, 3), f32)], axis=0)
    mask_p = jnp.concatenate(
        [mask.astype(f32), jnp.zeros((pe,), f32)], axis=0).reshape(E_PAD, 1)
    inda32 = inda.astype(jnp.int32)
    indb32 = indb.astype(jnp.int32)
    zpad = jnp.zeros((pe,), jnp.int32)
    npad = jnp.full((pe,), N, jnp.int32)
    inda_g = jnp.concatenate([inda32, zpad])
    indb_g = jnp.concatenate([indb32, zpad])
    inda_s = jnp.concatenate([inda32, npad])
    indb_s = jnp.concatenate([indb32, npad])

    species3 = species.astype(f32).reshape(N // BN, 1, BN)
    inde3 = inde.astype(f32).reshape(N // BN, 1, BN)

    wm4r = [W_m4[l].reshape(64, F, 5).transpose(0, 2, 1).reshape(64, 5 * F)
            for l in range(2)]
    wscr = [W_sc[l].reshape(S, 2 * F) for l in range(2)]

    w_node0 = (W_dn[0, 0], W_dn[0, 1], W_skip[0], wscr[0], W_post[0, 0],
               W_post[0, 1], W_ro0, W_up[1, 0], W_up[1, 1])
    w_e1 = (W_dn[1, 0], wscr[1], W_post[1, 0], W_mlp, W_ro1)

    tx0 = _k_init(species3, emb, W_up[0, 0])
    xs0 = _sc_gather(tx0, inda_g, 32)
    m0 = _k_edge_fwd(vecs_p, xs0, W_m1[0], W_m2[0], W_m3[0], wm4r[0], True)
    parts0 = _sc_scatter(m0, indb_s, 128)
    p0a, p0b = parts0[0], parts0[1]
    tx1, e0 = _k_node0_fwd(p0a, p0b, species3, w_node0)
    xs1 = _sc_gather(tx1, inda_g, 128)
    m1 = _k_edge_fwd(vecs_p, xs1, W_m1[1], W_m2[1], W_m3[1], wm4r[1], False)
    parts1 = _sc_scatter(m1, indb_s, 128)
    a1p0, a1p1 = parts1[0, :, 0:32], parts1[1, :, 0:32]
    eg = _k_final_e(a1p0, a1p1, species3, inde3, e0, w_e1)

    dagg1 = _k_node1_bwd(a1p0, a1p1, species3, w_e1)
    dm1 = _sc_gather(dagg1, indb_g, 32)
    dv1, dxs1 = _k_edge_bwd(vecs_p, xs1, dm1, W_m1[1], W_m2[1], W_m3[1],
                            wm4r[1], False, True)
    dtx = _sc_scatter(dxs1, inda_s, 128)
    dagg0 = _k_node0_bwd(p0a, p0b, species3, dtx[0], dtx[1], w_node0)
    dm0 = _sc_gather(dagg0, indb_g, 128)
    (dv0,) = _k_edge_bwd(vecs_p, xs0, dm0, W_m1[0], W_m2[0], W_m3[0],
                         wm4r[0], True, False)

    va, vb = _k_fo_vals(dv0, dv1, mask_p)
    vals2 = jnp.concatenate([va, vb], axis=0)
    idx2 = jnp.concatenate([inda_s, indb_s])
    fparts = _sc_scatter(vals2, idx2, 16)
    fo = (fparts[0] + fparts[1])[:N, 0:3]

    return eg[0], fo


# speedup vs baseline: 6.5494x; 1.3907x over previous
import math

import jax
import jax.numpy as jnp
import numpy as np
from jax.experimental import pallas as pl
from jax.experimental.pallas import tpu as pltpu
from jax.experimental.pallas import tpu_sc as plsc

N = 10000
E = 160000
F = 32
S = 8
G = 64
RMAX = 5.0
EPS = 0.1
_SQS = math.sqrt(float(S))
_OFFS = np.arange(0.0, -8.0, -1.0, dtype=np.float32).reshape(8, 1)

NC = 2
NS = 16
NW = NC * NS
CH = 128
N_PAD = 10240
E_PAD = 163840

BN = 1000
BE = 2048
BEB = 512

_SC_PARAMS = pltpu.CompilerParams(use_tc_tiling_on_sc=False)


def _pcall(f, grid, in_specs, out_specs, out_shape):
    return pl.pallas_call(f, grid=grid, in_specs=in_specs,
                          out_specs=out_specs, out_shape=out_shape)


def _sc_gather(table, idx, width):
    rows_per_w = E_PAD // NW
    nch = rows_per_w // CH
    mesh = plsc.VectorSubcoreMesh(core_axis_name="c", subcore_axis_name="s")

    def body(table_hbm, idx_hbm, out_hbm, idx_v, rows_v, sem):
        wid = jax.lax.axis_index("s") * NC + jax.lax.axis_index("c")
        base = wid * rows_per_w

        def step(i, carry):
            off = base + i * CH
            pltpu.sync_copy(idx_hbm.at[pl.ds(off, CH)], idx_v)
            pltpu.async_copy(table_hbm.at[idx_v], rows_v, sem).wait()
            pltpu.sync_copy(rows_v, out_hbm.at[pl.ds(off, CH)])
            return carry

        jax.lax.fori_loop(0, nch, step, 0)

    run = pl.kernel(
        body,
        out_type=jax.ShapeDtypeStruct((E_PAD, width), jnp.float32),
        mesh=mesh,
        compiler_params=_SC_PARAMS,
        scratch_types=[
            pltpu.VMEM((CH,), jnp.int32),
            pltpu.VMEM((CH, width), jnp.float32),
            pltpu.SemaphoreType.DMA,
        ],
    )
    return run(table, idx)


def _sc_scatter(vals, idx, width):
    vrows = vals.shape[0]
    rows_per_w = vrows // NW
    nch = rows_per_w // CH
    rows_per_sub = N_PAD // NS
    zeros = jnp.zeros((N_PAD, width), jnp.float32)
    mesh = plsc.VectorSubcoreMesh(core_axis_name="c", subcore_axis_name="s")

    def body(vals_hbm, idx_hbm, zeros_hbm, out_hbm, idx_v, vals_v, shared):
        cid = jax.lax.axis_index("c")
        sid = jax.lax.axis_index("s")
        wid = sid * NC + cid
        pltpu.sync_copy(zeros_hbm.at[pl.ds(sid * rows_per_sub, rows_per_sub)],
                        shared.at[pl.ds(sid * rows_per_sub, rows_per_sub)])
        plsc.subcore_barrier()
        base = wid * rows_per_w

        def step(i, carry):
            off = base + i * CH
            pltpu.sync_copy(idx_hbm.at[pl.ds(off, CH)], idx_v)
            pltpu.sync_copy(vals_hbm.at[pl.ds(off, CH)], vals_v)
            pltpu.sync_copy(vals_v, shared.at[idx_v], add=True)
            return carry

        jax.lax.fori_loop(0, nch, step, 0)
        plsc.subcore_barrier()
        pltpu.sync_copy(shared.at[pl.ds(sid * rows_per_sub, rows_per_sub)],
                        out_hbm.at[cid, pl.ds(sid * rows_per_sub, rows_per_sub)])

    run = pl.kernel(
        body,
        out_type=jax.ShapeDtypeStruct((NC, N_PAD, width), jnp.float32),
        mesh=mesh,
        compiler_params=_SC_PARAMS,
        scratch_types=[
            pltpu.VMEM((CH,), jnp.int32),
            pltpu.VMEM((CH, width), jnp.float32),
            pltpu.VMEM_SHARED((N_PAD, width), jnp.float32),
        ],
    )
    return run(vals, idx, zeros)


def _dot(a, b):
    return jnp.dot(a, b, preferred_element_type=jnp.float32)


def _split4(a):
    return a[:, 0:32], a[:, 32:64], a[:, 64:96], a[:, 96:128]


def _edge_math(vecs, xs, wm1, wm2, wm3, wm4r):
    vx, vy, vz = vecs[:, 0:1], vecs[:, 1:2], vecs[:, 2:3]
    r = jnp.sqrt(vx * vx + vy * vy + vz * vz)
    xr = jnp.maximum(r, 1e-9)
    ux, uy, uz = vx / xr, vy / xr, vz / xr
    ns = jax.lax.broadcasted_iota(jnp.int32, (1, 8), 1).astype(
        jnp.float32) + 1.0
    b = np.float32(np.sqrt(2.0 / RMAX)) * jnp.sin(ns * (np.pi / RMAX) * xr) / xr
    t = r * (1.0 / RMAX)
    xp = t * t * t * t * t
    env = 1.0 - 21.0 * xp + 35.0 * xp * t - 15.0 * xp * t * t
    cut = jnp.where(r < RMAX, env, 0.0)
    rad = b * cut
    h = jax.nn.silu(_dot(rad, wm1))
    h = jax.nn.silu(_dot(h, wm2))
    h = jax.nn.silu(_dot(h, wm3))
    mixr = _dot(h, wm4r)
    mix0, mix1, mix2, mix3, mix4 = (mixr[:, 32 * j:32 * j + 32]
                                    for j in range(5))
    x0, x1, x2, x3 = _split4(xs)
    dotv = x1 * ux + x2 * uy + x3 * uz
    m0 = mix0 * x0 + mix1 * dotv
    c1 = x2 * uz - x3 * uy
    c2 = x3 * ux - x1 * uz
    c3 = x1 * uy - x2 * ux
    mv1 = mix2 * x0 * ux + mix3 * x1 + mix4 * c1
    mv2 = mix2 * x0 * uy + mix3 * x2 + mix4 * c2
    mv3 = mix2 * x0 * uz + mix3 * x3 + mix4 * c3
    return jnp.concatenate([m0, mv1, mv2, mv3], axis=1) * EPS


def _edge_geom(vecs):
    vx, vy, vz = vecs[:, 0:1], vecs[:, 1:2], vecs[:, 2:3]
    r = jnp.sqrt(vx * vx + vy * vy + vz * vz)
    xr = jnp.maximum(r, 1e-9)
    ns = jax.lax.broadcasted_iota(jnp.int32, (1, 8), 1).astype(
        jnp.float32) + 1.0
    b = np.float32(np.sqrt(2.0 / RMAX)) * jnp.sin(ns * (np.pi / RMAX) * xr) / xr
    t = r * (1.0 / RMAX)
    xp = t * t * t * t * t
    env = 1.0 - 21.0 * xp + 35.0 * xp * t - 15.0 * xp * t * t
    rad = b * jnp.where(r < RMAX, env, 0.0)
    return rad, vx / xr, vy / xr, vz / xr


def _edge_mlp(rad, wm1, wm2, wm3, wm4):
    h = jax.nn.silu(_dot(rad, wm1))
    h = jax.nn.silu(_dot(h, wm2))
    h = jax.nn.silu(_dot(h, wm3))
    return _dot(h, wm4)


def _edge_math0(vecs, x0, wm1, wm2, wm3, wm4a):
    rad, ux, uy, uz = _edge_geom(vecs)
    mm = _edge_mlp(rad, wm1, wm2, wm3, wm4a)
    mix0, mix2 = mm[:, 0:32], mm[:, 32:64]
    s = mix2 * x0
    return jnp.concatenate(
        [mix0 * x0, s * ux, s * uy, s * uz], axis=1) * EPS


def _edge_math1_m0(vecs, xs, wm1, wm2, wm3, wm4b):
    rad, ux, uy, uz = _edge_geom(vecs)
    mm = _edge_mlp(rad, wm1, wm2, wm3, wm4b)
    mix0, mix1 = mm[:, 0:32], mm[:, 32:64]
    x0, x1, x2, x3 = _split4(xs)
    dotv = x1 * ux + x2 * uy + x3 * uz
    return (mix0 * x0 + mix1 * dotv) * EPS


def _node0_math(agg, oh, wdn0, wdn1, wskip0, wsc0r, wpost0, wpost1, wro0,
                wup10, wup11):
    a0, a1, a2, a3 = _split4(agg)
    y0 = _dot(a0, wdn0)
    yv = [_dot(a, wdn1) for a in (a1, a2, a3)]

    def gcontract(tq):
        acc = jnp.zeros_like(tq)
        for s in range(S):
            acc = acc + oh[:, s:s + 1] * _dot(tq, wskip0[s])
        return acc * (1.0 / _SQS)

    y0 = gcontract(y0)
    yv = [gcontract(y) for y in yv]
    ws = _dot(oh, wsc0r)
    ws0, ws1 = ws[:, 0:32], ws[:, 32:64]
    z0 = ws0 * y0 + ws1 * y0 * y0
    p0 = _dot(z0, wpost0)
    pv = [_dot(ws0 * y, wpost1) for y in yv]
    e0 = _dot(p0, wro0)
    tx1 = jnp.concatenate([_dot(p0, wup10)] + [_dot(p, wup11) for p in pv],
                          axis=1)
    return tx1, e0


def _e1_math(a0, oh, wdn10, wsc1r, wpost10, wmlp, wro1):
    y0 = _dot(a0, wdn10)
    ws = _dot(oh, wsc1r)
    z0 = ws[:, 0:32] * y0 + ws[:, 32:64] * y0 * y0
    p0 = _dot(z0, wpost10)
    h = jax.nn.silu(_dot(p0, wmlp))
    return _dot(h, wro1)


def _onehot(sp_ref, k):
    sp = sp_ref[0, 0, :]
    ids = jax.lax.broadcasted_iota(jnp.int32, (sp.shape[0], k), 1).astype(
        jnp.float32)
    return jnp.where(sp[:, None] == ids, 1.0, 0.0)


def _full(shape):
    return pl.BlockSpec(shape, lambda i: tuple(0 for _ in shape))


def _rows(bs, w):
    return pl.BlockSpec((bs, w), lambda i: (i, 0))


def _sp3(bs):
    return pl.BlockSpec((1, 1, bs), lambda i: (i, 0, 0))


def _k_init(species3, emb, wup00):
    def body(sp_ref, emb_ref, w_ref, tx_ref):
        oh = _onehot(sp_ref, S)
        x00 = _dot(oh, emb_ref[...]) * (1.0 / _SQS)
        tx_ref[...] = _dot(x00, w_ref[...])

    return _pcall(body, (N // BN,),
                  [_sp3(BN), _full((S, F)), _full((F, F))],
                  _rows(BN, 32),
                  jax.ShapeDtypeStruct((N, 32), jnp.float32))(
                      species3, emb, wup00)


def _k_edge_fwd(vecs_p, xs, wm1, wm2, wm3, wm4r):
    def body(v_ref, xs_ref, w1, w2, w3, w4, m_ref):
        m_ref[...] = _edge_math(v_ref[...], xs_ref[...], w1[...], w2[...],
                                w3[...], w4[...])

    return _pcall(body, (E_PAD // BE,),
                  [_rows(BE, 3), _rows(BE, 128), _full((8, 64)),
                   _full((64, 64)), _full((64, 64)), _full((64, 160))],
                  _rows(BE, 128),
                  jax.ShapeDtypeStruct((E_PAD, 128), jnp.float32))(
                      vecs_p, xs, wm1, wm2, wm3, wm4r)


def _k_edge_fwd0(vecs_p, xs0, wm1, wm2, wm3, wm4a):
    def body(v_ref, xs_ref, w1, w2, w3, w4, m_ref):
        m_ref[...] = _edge_math0(v_ref[...], xs_ref[...], w1[...], w2[...],
                                 w3[...], w4[...])

    return _pcall(body, (E_PAD // BE,),
                  [_rows(BE, 3), _rows(BE, 32), _full((8, 64)),
                   _full((64, 64)), _full((64, 64)), _full((64, 64))],
                  _rows(BE, 128),
                  jax.ShapeDtypeStruct((E_PAD, 128), jnp.float32))(
                      vecs_p, xs0, wm1, wm2, wm3, wm4a)


def _k_edge_bwd1(vecs_p, xs, dm32, wm1, wm2, wm3, wm4b):
    def body(v_ref, xs_ref, dm_ref, w1, w2, w3, w4, dv_ref, dxs_ref):
        w1v, w2v, w3v, w4v = w1[...], w2[...], w3[...], w4[...]
        fn = lambda v, x: _edge_math1_m0(v, x, w1v, w2v, w3v, w4v)
        _, vjpf = jax.vjp(fn, v_ref[...], xs_ref[...])
        dv, dxs = vjpf(dm_ref[...])
        dv_ref[...] = dv
        dxs_ref[...] = dxs

    return _pcall(body, (E_PAD // BEB,),
                  [_rows(BEB, 3), _rows(BEB, 128), _rows(BEB, 32),
                   _full((8, 64)), _full((64, 64)), _full((64, 64)),
                   _full((64, 64))],
                  [_rows(BEB, 3), _rows(BEB, 128)],
                  [jax.ShapeDtypeStruct((E_PAD, 3), jnp.float32),
                   jax.ShapeDtypeStruct((E_PAD, 128), jnp.float32)])(
                      vecs_p, xs, dm32, wm1, wm2, wm3, wm4b)


def _k_edge_bwd0(vecs_p, xs0, dm, wm1, wm2, wm3, wm4a):
    def body(v_ref, xs_ref, dm_ref, w1, w2, w3, w4, dv_ref):
        w1v, w2v, w3v, w4v = w1[...], w2[...], w3[...], w4[...]
        x0v = xs_ref[...]
        fn = lambda v: _edge_math0(v, x0v, w1v, w2v, w3v, w4v)
        _, vjpf = jax.vjp(fn, v_ref[...])
        (dv,) = vjpf(dm_ref[...])
        dv_ref[...] = dv

    return _pcall(body, (E_PAD // BEB,),
                  [_rows(BEB, 3), _rows(BEB, 32), _rows(BEB, 128),
                   _full((8, 64)), _full((64, 64)), _full((64, 64)),
                   _full((64, 64))],
                  _rows(BEB, 3),
                  jax.ShapeDtypeStruct((E_PAD, 3), jnp.float32))(
                      vecs_p, xs0, dm, wm1, wm2, wm3, wm4a)


def _k_node0_fwd(p0t, p1t, species3, w):
    def body(p0_ref, p1_ref, sp_ref, wdn0, wdn1, wskip0, wsc0r, wpost0,
             wpost1, wro0, wup10, wup11, tx_ref, e_ref):
        agg = p0_ref[...] + p1_ref[...]
        oh = _onehot(sp_ref, S)
        tx1, e0 = _node0_math(agg, oh, wdn0[...], wdn1[...], wskip0[...],
                              wsc0r[...], wpost0[...], wpost1[...], wro0[...],
                              wup10[...], wup11[...])
        tx_ref[...] = tx1
        e_ref[...] = e0

    return _pcall(body, (N // BN,),
                  [_rows(BN, 128), _rows(BN, 128), _sp3(BN),
                   _full((F, F)), _full((F, F)), _full((S, F, F)),
                   _full((S, 2 * F)), _full((F, F)), _full((F, F)),
                   _full((F, 1)), _full((F, F)), _full((F, F))],
                  [_rows(BN, 128), _rows(BN, 1)],
                  [jax.ShapeDtypeStruct((N, 128), jnp.float32),
                   jax.ShapeDtypeStruct((N, 1), jnp.float32)])(
                      p0t, p1t, species3, *w)


def _k_node0_bwd(p0t, p1t, species3, dtx0, dtx1, w):
    def body(p0_ref, p1_ref, sp_ref, dt0_ref, dt1_ref, wdn0, wdn1, wskip0,
             wsc0r, wpost0, wpost1, wro0, wup10, wup11, dagg_ref):
        agg = p0_ref[...] + p1_ref[...]
        oh = _onehot(sp_ref, S)
        args = (wdn0[...], wdn1[...], wskip0[...], wsc0r[...], wpost0[...],
                wpost1[...], wro0[...], wup10[...], wup11[...])
        fn = lambda a: _node0_math(a, oh, *args)
        _, vjpf = jax.vjp(fn, agg)
        dtx = dt0_ref[...] + dt1_ref[...]
        (dagg,) = vjpf((dtx, jnp.ones((agg.shape[0], 1), jnp.float32)))
        dagg_ref[...] = dagg

    return _pcall(body, (N // BN,),
                  [_rows(BN, 128), _rows(BN, 128), _sp3(BN),
                   _rows(BN, 128), _rows(BN, 128),
                   _full((F, F)), _full((F, F)), _full((S, F, F)),
                   _full((S, 2 * F)), _full((F, F)), _full((F, F)),
                   _full((F, 1)), _full((F, F)), _full((F, F))],
                  _rows(BN, 128),
                  jax.ShapeDtypeStruct((N, 128), jnp.float32))(
                      p0t, p1t, species3, dtx0, dtx1, *w)


def _k_final_e(a0p0, a0p1, species3, inde3, e0, w):
    def body(p0_ref, p1_ref, sp_ref, ge_ref, e0_ref, wdn10, wsc1r, wpost10,
             wmlp, wro1, offs_ref, eg_ref):
        i = pl.program_id(0)
        a0 = p0_ref[...] + p1_ref[...]
        oh = _onehot(sp_ref, S)
        e1 = _e1_math(a0, oh, wdn10[...], wsc1r[...], wpost10[...],
                      wmlp[...], wro1[...])
        off = _dot(oh, offs_ref[...])
        ei = e0_ref[...] + e1 + off
        ohg = _onehot(ge_ref, G)
        blk = _dot(ei.reshape(1, ei.shape[0]), ohg)

        @pl.when(i == 0)
        def _():
            eg_ref[...] = jnp.zeros_like(eg_ref)

        eg_ref[...] += blk

    return _pcall(body, (N // BN,),
                  [_rows(BN, 32), _rows(BN, 32), _sp3(BN), _sp3(BN),
                   _rows(BN, 1), _full((F, F)), _full((S, 2 * F)),
                   _full((F, F)), _full((F, 16)), _full((16, 1)),
                   _full((S, 1))],
                  pl.BlockSpec((1, G), lambda i: (0, 0)),
                  jax.ShapeDtypeStruct((1, G), jnp.float32))(
                      a0p0, a0p1, species3, inde3, e0, *w,
                      jnp.asarray(_OFFS))


def _k_node1_bwd(a0p0, a0p1, species3, w):
    def body(p0_ref, p1_ref, sp_ref, wdn10, wsc1r, wpost10, wmlp, wro1,
             da_ref):
        a0 = p0_ref[...] + p1_ref[...]
        oh = _onehot(sp_ref, S)
        args = (wdn10[...], wsc1r[...], wpost10[...], wmlp[...], wro1[...])
        fn = lambda a: _e1_math(a, oh, *args)
        _, vjpf = jax.vjp(fn, a0)
        (da,) = vjpf(jnp.ones((a0.shape[0], 1), jnp.float32))
        da_ref[...] = da

    return _pcall(body, (N // BN,),
                  [_rows(BN, 32), _rows(BN, 32), _sp3(BN), _full((F, F)),
                   _full((S, 2 * F)), _full((F, F)), _full((F, 16)),
                   _full((16, 1))],
                  _rows(BN, 32),
                  jax.ShapeDtypeStruct((N, 32), jnp.float32))(
                      a0p0, a0p1, species3, *w)


def _k_fo_vals(dv0, dv1, mask_p):
    def body(d0_ref, d1_ref, m_ref, va_ref, vb_ref):
        ft = (d0_ref[...] + d1_ref[...]) * m_ref[...]
        pad = jnp.zeros((ft.shape[0], 13), jnp.float32)
        va = jnp.concatenate([ft, pad], axis=1)
        va_ref[...] = va
        vb_ref[...] = -va

    return _pcall(body, (E_PAD // BE,),
                  [_rows(BE, 3), _rows(BE, 3), _rows(BE, 1)],
                  [_rows(BE, 16), _rows(BE, 16)],
                  [jax.ShapeDtypeStruct((E_PAD, 16), jnp.float32),
                   jax.ShapeDtypeStruct((E_PAD, 16), jnp.float32)])(
                      dv0, dv1, mask_p)


def kernel(nn_vecs, species, inda, indb, inde, mask, nats, emb, W_up, W_m1,
           W_m2, W_m3, W_m4, W_dn, W_sc, W_post, W_skip, W_ro0, W_mlp,
           W_ro1):
    f32 = jnp.float32
    pe = E_PAD - E

    vecs_p = jnp.concatenate(
        [nn_vecs.astype(f32), jnp.ones((pe, 3), f32)], axis=0)
    mask_p = jnp.concatenate(
        [mask.astype(f32), jnp.zeros((pe,), f32)], axis=0).reshape(E_PAD, 1)
    inda32 = inda.astype(jnp.int32)
    indb32 = indb.astype(jnp.int32)
    zpad = jnp.zeros((pe,), jnp.int32)
    npad = jnp.full((pe,), N, jnp.int32)
    inda_g = jnp.concatenate([inda32, zpad])
    indb_g = jnp.concatenate([indb32, zpad])
    inda_s = jnp.concatenate([inda32, npad])
    indb_s = jnp.concatenate([indb32, npad])

    species3 = species.astype(f32).reshape(N // BN, 1, BN)
    inde3 = inde.astype(f32).reshape(N // BN, 1, BN)

    wm4r = [W_m4[l].reshape(64, F, 5).transpose(0, 2, 1).reshape(64, 5 * F)
            for l in range(2)]
    wscr = [W_sc[l].reshape(S, 2 * F) for l in range(2)]

    w_node0 = (W_dn[0, 0], W_dn[0, 1], W_skip[0], wscr[0], W_post[0, 0],
               W_post[0, 1], W_ro0, W_up[1, 0], W_up[1, 1])
    w_e1 = (W_dn[1, 0], wscr[1], W_post[1, 0], W_mlp, W_ro1)

    tx0 = _k_init(species3, emb, W_up[0, 0])
    xs0 = _sc_gather(tx0, inda_g, 32)
    wm4a0 = jnp.concatenate([wm4r[0][:, 0:32], wm4r[0][:, 64:96]], axis=1)
    wm4b1 = wm4r[1][:, 0:64]
    m0 = _k_edge_fwd0(vecs_p, xs0, W_m1[0], W_m2[0], W_m3[0], wm4a0)
    parts0 = _sc_scatter(m0, indb_s, 128)
    p0a, p0b = parts0[0], parts0[1]
    tx1, e0 = _k_node0_fwd(p0a, p0b, species3, w_node0)
    xs1 = _sc_gather(tx1, inda_g, 128)
    m1 = _k_edge_fwd(vecs_p, xs1, W_m1[1], W_m2[1], W_m3[1], wm4r[1])
    parts1 = _sc_scatter(m1, indb_s, 128)
    a1p0, a1p1 = parts1[0, :, 0:32], parts1[1, :, 0:32]
    eg = _k_final_e(a1p0, a1p1, species3, inde3, e0, w_e1)

    dagg1 = _k_node1_bwd(a1p0, a1p1, species3, w_e1)
    dm1 = _sc_gather(dagg1, indb_g, 32)
    dv1, dxs1 = _k_edge_bwd1(vecs_p, xs1, dm1, W_m1[1], W_m2[1], W_m3[1],
                             wm4b1)
    dtx = _sc_scatter(dxs1, inda_s, 128)
    dagg0 = _k_node0_bwd(p0a, p0b, species3, dtx[0], dtx[1], w_node0)
    dm0 = _sc_gather(dagg0, indb_g, 128)
    dv0 = _k_edge_bwd0(vecs_p, xs0, dm0, W_m1[0], W_m2[0], W_m3[0], wm4a0)

    va, vb = _k_fo_vals(dv0, dv1, mask_p)
    vals2 = jnp.concatenate([va, vb], axis=0)
    idx2 = jnp.concatenate([inda_s, indb_s])
    fparts = _sc_scatter(vals2, idx2, 16)
    fo = (fparts[0] + fparts[1])[:N, 0:3]

    return eg[0], fo


# speedup vs baseline: 6.8374x; 1.0440x over previous
import math

import jax
import jax.numpy as jnp
import numpy as np
from jax.experimental import pallas as pl
from jax.experimental.pallas import tpu as pltpu
from jax.experimental.pallas import tpu_sc as plsc

N = 10000
E = 160000
F = 32
S = 8
G = 64
RMAX = 5.0
EPS = 0.1
_SQS = math.sqrt(float(S))
_OFFS = np.arange(0.0, -8.0, -1.0, dtype=np.float32).reshape(8, 1)

NC = 2
NS = 16
NW = NC * NS
CH = 128
N_PAD = 10240
E_PAD = 163840

BN = 1000
BE = 2048
BEB = 1024

_SC_PARAMS = pltpu.CompilerParams(use_tc_tiling_on_sc=False)


def _pcall(f, grid, in_specs, out_specs, out_shape):
    return pl.pallas_call(f, grid=grid, in_specs=in_specs,
                          out_specs=out_specs, out_shape=out_shape)


def _sc_gather(table, idx, width):
    rows_per_w = E_PAD // NW
    nch = rows_per_w // CH
    mesh = plsc.VectorSubcoreMesh(core_axis_name="c", subcore_axis_name="s")

    def body(table_hbm, idx_hbm, out_hbm, idx_v, rows_v, sem):
        wid = jax.lax.axis_index("s") * NC + jax.lax.axis_index("c")
        base = wid * rows_per_w

        def step(i, carry):
            off = base + i * CH
            pltpu.sync_copy(idx_hbm.at[pl.ds(off, CH)], idx_v)
            pltpu.async_copy(table_hbm.at[idx_v], rows_v, sem).wait()
            pltpu.sync_copy(rows_v, out_hbm.at[pl.ds(off, CH)])
            return carry

        jax.lax.fori_loop(0, nch, step, 0)

    run = pl.kernel(
        body,
        out_type=jax.ShapeDtypeStruct((E_PAD, width), jnp.float32),
        mesh=mesh,
        compiler_params=_SC_PARAMS,
        scratch_types=[
            pltpu.VMEM((CH,), jnp.int32),
            pltpu.VMEM((CH, width), jnp.float32),
            pltpu.SemaphoreType.DMA,
        ],
    )
    return run(table, idx)


def _sc_scatter(vals, idx, width):
    vrows = vals.shape[0]
    rows_per_w = vrows // NW
    nch = rows_per_w // CH
    rows_per_sub = N_PAD // NS
    zeros = jnp.zeros((N_PAD, width), jnp.float32)
    mesh = plsc.VectorSubcoreMesh(core_axis_name="c", subcore_axis_name="s")

    def body(vals_hbm, idx_hbm, zeros_hbm, out_hbm, idx_v, vals_v, shared):
        cid = jax.lax.axis_index("c")
        sid = jax.lax.axis_index("s")
        wid = sid * NC + cid
        pltpu.sync_copy(zeros_hbm.at[pl.ds(sid * rows_per_sub, rows_per_sub)],
                        shared.at[pl.ds(sid * rows_per_sub, rows_per_sub)])
        plsc.subcore_barrier()
        base = wid * rows_per_w

        def step(i, carry):
            off = base + i * CH
            pltpu.sync_copy(idx_hbm.at[pl.ds(off, CH)], idx_v)
            pltpu.sync_copy(vals_hbm.at[pl.ds(off, CH)], vals_v)
            pltpu.sync_copy(vals_v, shared.at[idx_v], add=True)
            return carry

        jax.lax.fori_loop(0, nch, step, 0)
        plsc.subcore_barrier()
        pltpu.sync_copy(shared.at[pl.ds(sid * rows_per_sub, rows_per_sub)],
                        out_hbm.at[cid, pl.ds(sid * rows_per_sub, rows_per_sub)])

    run = pl.kernel(
        body,
        out_type=jax.ShapeDtypeStruct((NC, N_PAD, width), jnp.float32),
        mesh=mesh,
        compiler_params=_SC_PARAMS,
        scratch_types=[
            pltpu.VMEM((CH,), jnp.int32),
            pltpu.VMEM((CH, width), jnp.float32),
            pltpu.VMEM_SHARED((N_PAD, width), jnp.float32),
        ],
    )
    return run(vals, idx, zeros)


def _dot(a, b):
    return jnp.dot(a, b, preferred_element_type=jnp.float32)


def _split4(a):
    return a[:, 0:32], a[:, 32:64], a[:, 64:96], a[:, 96:128]


def _edge_math(vecs, xs, wm1, wm2, wm3, wm4r):
    vx, vy, vz = vecs[:, 0:1], vecs[:, 1:2], vecs[:, 2:3]
    r = jnp.sqrt(vx * vx + vy * vy + vz * vz)
    xr = jnp.maximum(r, 1e-9)
    ux, uy, uz = vx / xr, vy / xr, vz / xr
    ns = jax.lax.broadcasted_iota(jnp.int32, (1, 8), 1).astype(
        jnp.float32) + 1.0
    b = np.float32(np.sqrt(2.0 / RMAX)) * jnp.sin(ns * (np.pi / RMAX) * xr) / xr
    t = r * (1.0 / RMAX)
    xp = t * t * t * t * t
    env = 1.0 - 21.0 * xp + 35.0 * xp * t - 15.0 * xp * t * t
    cut = jnp.where(r < RMAX, env, 0.0)
    rad = b * cut
    h = jax.nn.silu(_dot(rad, wm1))
    h = jax.nn.silu(_dot(h, wm2))
    h = jax.nn.silu(_dot(h, wm3))
    mixr = _dot(h, wm4r)
    mix0, mix1, mix2, mix3, mix4 = (mixr[:, 32 * j:32 * j + 32]
                                    for j in range(5))
    x0, x1, x2, x3 = _split4(xs)
    dotv = x1 * ux + x2 * uy + x3 * uz
    m0 = mix0 * x0 + mix1 * dotv
    c1 = x2 * uz - x3 * uy
    c2 = x3 * ux - x1 * uz
    c3 = x1 * uy - x2 * ux
    mv1 = mix2 * x0 * ux + mix3 * x1 + mix4 * c1
    mv2 = mix2 * x0 * uy + mix3 * x2 + mix4 * c2
    mv3 = mix2 * x0 * uz + mix3 * x3 + mix4 * c3
    return jnp.concatenate([m0, mv1, mv2, mv3], axis=1) * EPS


def _edge_geom(vecs):
    vx, vy, vz = vecs[:, 0:1], vecs[:, 1:2], vecs[:, 2:3]
    r = jnp.sqrt(vx * vx + vy * vy + vz * vz)
    xr = jnp.maximum(r, 1e-9)
    ns = jax.lax.broadcasted_iota(jnp.int32, (1, 8), 1).astype(
        jnp.float32) + 1.0
    b = np.float32(np.sqrt(2.0 / RMAX)) * jnp.sin(ns * (np.pi / RMAX) * xr) / xr
    t = r * (1.0 / RMAX)
    xp = t * t * t * t * t
    env = 1.0 - 21.0 * xp + 35.0 * xp * t - 15.0 * xp * t * t
    rad = b * jnp.where(r < RMAX, env, 0.0)
    return rad, vx / xr, vy / xr, vz / xr


def _edge_mlp(rad, wm1, wm2, wm3, wm4):
    h = jax.nn.silu(_dot(rad, wm1))
    h = jax.nn.silu(_dot(h, wm2))
    h = jax.nn.silu(_dot(h, wm3))
    return _dot(h, wm4)


def _edge_math0(vecs, x0, wm1, wm2, wm3, wm4a):
    rad, ux, uy, uz = _edge_geom(vecs)
    mm = _edge_mlp(rad, wm1, wm2, wm3, wm4a)
    mix0, mix2 = mm[:, 0:32], mm[:, 32:64]
    s = mix2 * x0
    return jnp.concatenate(
        [mix0 * x0, s * ux, s * uy, s * uz], axis=1) * EPS


def _edge_math1_m0(vecs, xs, wm1, wm2, wm3, wm4b):
    rad, ux, uy, uz = _edge_geom(vecs)
    mm = _edge_mlp(rad, wm1, wm2, wm3, wm4b)
    mix0, mix1 = mm[:, 0:32], mm[:, 32:64]
    x0, x1, x2, x3 = _split4(xs)
    dotv = x1 * ux + x2 * uy + x3 * uz
    return (mix0 * x0 + mix1 * dotv) * EPS


def _node0_math(agg, oh, wdn0, wdn1, wskip0, wsc0r, wpost0, wpost1, wro0,
                wup10, wup11):
    a0, a1, a2, a3 = _split4(agg)
    y0 = _dot(a0, wdn0)
    yv = [_dot(a, wdn1) for a in (a1, a2, a3)]

    def gcontract(tq):
        acc = jnp.zeros_like(tq)
        for s in range(S):
            acc = acc + oh[:, s:s + 1] * _dot(tq, wskip0[s])
        return acc * (1.0 / _SQS)

    y0 = gcontract(y0)
    yv = [gcontract(y) for y in yv]
    ws = _dot(oh, wsc0r)
    ws0, ws1 = ws[:, 0:32], ws[:, 32:64]
    z0 = ws0 * y0 + ws1 * y0 * y0
    p0 = _dot(z0, wpost0)
    pv = [_dot(ws0 * y, wpost1) for y in yv]
    e0 = _dot(p0, wro0)
    tx1 = jnp.concatenate([_dot(p0, wup10)] + [_dot(p, wup11) for p in pv],
                          axis=1)
    return tx1, e0


def _e1_math(a0, oh, wdn10, wsc1r, wpost10, wmlp, wro1):
    y0 = _dot(a0, wdn10)
    ws = _dot(oh, wsc1r)
    z0 = ws[:, 0:32] * y0 + ws[:, 32:64] * y0 * y0
    p0 = _dot(z0, wpost10)
    h = jax.nn.silu(_dot(p0, wmlp))
    return _dot(h, wro1)


def _onehot(sp_ref, k):
    sp = sp_ref[0, 0, :]
    ids = jax.lax.broadcasted_iota(jnp.int32, (sp.shape[0], k), 1).astype(
        jnp.float32)
    return jnp.where(sp[:, None] == ids, 1.0, 0.0)


def _full(shape):
    return pl.BlockSpec(shape, lambda i: tuple(0 for _ in shape))


def _rows(bs, w):
    return pl.BlockSpec((bs, w), lambda i: (i, 0))


def _sp3(bs):
    return pl.BlockSpec((1, 1, bs), lambda i: (i, 0, 0))


def _k_init(species3, emb, wup00):
    def body(sp_ref, emb_ref, w_ref, tx_ref):
        oh = _onehot(sp_ref, S)
        x00 = _dot(oh, emb_ref[...]) * (1.0 / _SQS)
        tx_ref[...] = _dot(x00, w_ref[...])

    return _pcall(body, (N // BN,),
                  [_sp3(BN), _full((S, F)), _full((F, F))],
                  _rows(BN, 32),
                  jax.ShapeDtypeStruct((N, 32), jnp.float32))(
                      species3, emb, wup00)


def _k_edge_fwd(vecs_p, xs, wm1, wm2, wm3, wm4r):
    def body(v_ref, xs_ref, w1, w2, w3, w4, m_ref):
        m_ref[...] = _edge_math(v_ref[...], xs_ref[...], w1[...], w2[...],
                                w3[...], w4[...])

    return _pcall(body, (E_PAD // BE,),
                  [_rows(BE, 3), _rows(BE, 128), _full((8, 64)),
                   _full((64, 64)), _full((64, 64)), _full((64, 160))],
                  _rows(BE, 128),
                  jax.ShapeDtypeStruct((E_PAD, 128), jnp.float32))(
                      vecs_p, xs, wm1, wm2, wm3, wm4r)


def _k_edge_fwd0(vecs_p, xs0, wm1, wm2, wm3, wm4a):
    def body(v_ref, xs_ref, w1, w2, w3, w4, m_ref):
        m_ref[...] = _edge_math0(v_ref[...], xs_ref[...], w1[...], w2[...],
                                 w3[...], w4[...])

    return _pcall(body, (E_PAD // BE,),
                  [_rows(BE, 3), _rows(BE, 32), _full((8, 64)),
                   _full((64, 64)), _full((64, 64)), _full((64, 64))],
                  _rows(BE, 128),
                  jax.ShapeDtypeStruct((E_PAD, 128), jnp.float32))(
                      vecs_p, xs0, wm1, wm2, wm3, wm4a)


def _k_edge_bwd1(vecs_p, xs, dm32, wm1, wm2, wm3, wm4b):
    def body(v_ref, xs_ref, dm_ref, w1, w2, w3, w4, dv_ref, dxs_ref):
        w1v, w2v, w3v, w4v = w1[...], w2[...], w3[...], w4[...]
        fn = lambda v, x: _edge_math1_m0(v, x, w1v, w2v, w3v, w4v)
        _, vjpf = jax.vjp(fn, v_ref[...], xs_ref[...])
        dv, dxs = vjpf(dm_ref[...])
        dv_ref[...] = dv
        dxs_ref[...] = dxs

    return _pcall(body, (E_PAD // BEB,),
                  [_rows(BEB, 3), _rows(BEB, 128), _rows(BEB, 32),
                   _full((8, 64)), _full((64, 64)), _full((64, 64)),
                   _full((64, 64))],
                  [_rows(BEB, 3), _rows(BEB, 128)],
                  [jax.ShapeDtypeStruct((E_PAD, 3), jnp.float32),
                   jax.ShapeDtypeStruct((E_PAD, 128), jnp.float32)])(
                      vecs_p, xs, dm32, wm1, wm2, wm3, wm4b)


def _k_edge_bwd0(vecs_p, xs0, dm, wm1, wm2, wm3, wm4a):
    def body(v_ref, xs_ref, dm_ref, w1, w2, w3, w4, dv_ref):
        w1v, w2v, w3v, w4v = w1[...], w2[...], w3[...], w4[...]
        x0v = xs_ref[...]
        fn = lambda v: _edge_math0(v, x0v, w1v, w2v, w3v, w4v)
        _, vjpf = jax.vjp(fn, v_ref[...])
        (dv,) = vjpf(dm_ref[...])
        dv_ref[...] = dv

    return _pcall(body, (E_PAD // BEB,),
                  [_rows(BEB, 3), _rows(BEB, 32), _rows(BEB, 128),
                   _full((8, 64)), _full((64, 64)), _full((64, 64)),
                   _full((64, 64))],
                  _rows(BEB, 3),
                  jax.ShapeDtypeStruct((E_PAD, 3), jnp.float32))(
                      vecs_p, xs0, dm, wm1, wm2, wm3, wm4a)


def _k_node0_fwd(p0t, p1t, species3, w):
    def body(p0_ref, p1_ref, sp_ref, wdn0, wdn1, wskip0, wsc0r, wpost0,
             wpost1, wro0, wup10, wup11, tx_ref, e_ref):
        agg = p0_ref[...] + p1_ref[...]
        oh = _onehot(sp_ref, S)
        tx1, e0 = _node0_math(agg, oh, wdn0[...], wdn1[...], wskip0[...],
                              wsc0r[...], wpost0[...], wpost1[...], wro0[...],
                              wup10[...], wup11[...])
        tx_ref[...] = tx1
        e_ref[...] = e0

    return _pcall(body, (N // BN,),
                  [_rows(BN, 128), _rows(BN, 128), _sp3(BN),
                   _full((F, F)), _full((F, F)), _full((S, F, F)),
                   _full((S, 2 * F)), _full((F, F)), _full((F, F)),
                   _full((F, 1)), _full((F, F)), _full((F, F))],
                  [_rows(BN, 128), _rows(BN, 1)],
                  [jax.ShapeDtypeStruct((N, 128), jnp.float32),
                   jax.ShapeDtypeStruct((N, 1), jnp.float32)])(
                      p0t, p1t, species3, *w)


def _k_node0_bwd(p0t, p1t, species3, dtx0, dtx1, w):
    def body(p0_ref, p1_ref, sp_ref, dt0_ref, dt1_ref, wdn0, wdn1, wskip0,
             wsc0r, wpost0, wpost1, wro0, wup10, wup11, dagg_ref):
        agg = p0_ref[...] + p1_ref[...]
        oh = _onehot(sp_ref, S)
        args = (wdn0[...], wdn1[...], wskip0[...], wsc0r[...], wpost0[...],
                wpost1[...], wro0[...], wup10[...], wup11[...])
        fn = lambda a: _node0_math(a, oh, *args)
        _, vjpf = jax.vjp(fn, agg)
        dtx = dt0_ref[...] + dt1_ref[...]
        (dagg,) = vjpf((dtx, jnp.ones((agg.shape[0], 1), jnp.float32)))
        dagg_ref[...] = dagg

    return _pcall(body, (N // BN,),
                  [_rows(BN, 128), _rows(BN, 128), _sp3(BN),
                   _rows(BN, 128), _rows(BN, 128),
                   _full((F, F)), _full((F, F)), _full((S, F, F)),
                   _full((S, 2 * F)), _full((F, F)), _full((F, F)),
                   _full((F, 1)), _full((F, F)), _full((F, F))],
                  _rows(BN, 128),
                  jax.ShapeDtypeStruct((N, 128), jnp.float32))(
                      p0t, p1t, species3, dtx0, dtx1, *w)


def _k_final_e(a0p0, a0p1, species3, inde3, e0, w):
    def body(p0_ref, p1_ref, sp_ref, ge_ref, e0_ref, wdn10, wsc1r, wpost10,
             wmlp, wro1, offs_ref, eg_ref):
        i = pl.program_id(0)
        a0 = p0_ref[...] + p1_ref[...]
        oh = _onehot(sp_ref, S)
        e1 = _e1_math(a0, oh, wdn10[...], wsc1r[...], wpost10[...],
                      wmlp[...], wro1[...])
        off = _dot(oh, offs_ref[...])
        ei = e0_ref[...] + e1 + off
        ohg = _onehot(ge_ref, G)
        blk = _dot(ei.reshape(1, ei.shape[0]), ohg)

        @pl.when(i == 0)
        def _():
            eg_ref[...] = jnp.zeros_like(eg_ref)

        eg_ref[...] += blk

    return _pcall(body, (N // BN,),
                  [_rows(BN, 32), _rows(BN, 32), _sp3(BN), _sp3(BN),
                   _rows(BN, 1), _full((F, F)), _full((S, 2 * F)),
                   _full((F, F)), _full((F, 16)), _full((16, 1)),
                   _full((S, 1))],
                  pl.BlockSpec((1, G), lambda i: (0, 0)),
                  jax.ShapeDtypeStruct((1, G), jnp.float32))(
                      a0p0, a0p1, species3, inde3, e0, *w,
                      jnp.asarray(_OFFS))


def _k_node1_bwd(a0p0, a0p1, species3, w):
    def body(p0_ref, p1_ref, sp_ref, wdn10, wsc1r, wpost10, wmlp, wro1,
             da_ref):
        a0 = p0_ref[...] + p1_ref[...]
        oh = _onehot(sp_ref, S)
        args = (wdn10[...], wsc1r[...], wpost10[...], wmlp[...], wro1[...])
        fn = lambda a: _e1_math(a, oh, *args)
        _, vjpf = jax.vjp(fn, a0)
        (da,) = vjpf(jnp.ones((a0.shape[0], 1), jnp.float32))
        da_ref[...] = da

    return _pcall(body, (N // BN,),
                  [_rows(BN, 32), _rows(BN, 32), _sp3(BN), _full((F, F)),
                   _full((S, 2 * F)), _full((F, F)), _full((F, 16)),
                   _full((16, 1))],
                  _rows(BN, 32),
                  jax.ShapeDtypeStruct((N, 32), jnp.float32))(
                      a0p0, a0p1, species3, *w)


def _k_fo_vals(dv0, dv1, mask_p):
    def body(d0_ref, d1_ref, m_ref, va_ref, vb_ref):
        ft = (d0_ref[...] + d1_ref[...]) * m_ref[...]
        pad = jnp.zeros((ft.shape[0], 13), jnp.float32)
        va = jnp.concatenate([ft, pad], axis=1)
        va_ref[...] = va
        vb_ref[...] = -va

    return _pcall(body, (E_PAD // BE,),
                  [_rows(BE, 3), _rows(BE, 3), _rows(BE, 1)],
                  [_rows(BE, 16), _rows(BE, 16)],
                  [jax.ShapeDtypeStruct((E_PAD, 16), jnp.float32),
                   jax.ShapeDtypeStruct((E_PAD, 16), jnp.float32)])(
                      dv0, dv1, mask_p)


def kernel(nn_vecs, species, inda, indb, inde, mask, nats, emb, W_up, W_m1,
           W_m2, W_m3, W_m4, W_dn, W_sc, W_post, W_skip, W_ro0, W_mlp,
           W_ro1):
    f32 = jnp.float32
    pe = E_PAD - E

    vecs_p = jnp.concatenate(
        [nn_vecs.astype(f32), jnp.ones((pe, 3), f32)], axis=0)
    mask_p = jnp.concatenate(
        [mask.astype(f32), jnp.zeros((pe,), f32)], axis=0).reshape(E_PAD, 1)
    inda32 = inda.astype(jnp.int32)
    indb32 = indb.astype(jnp.int32)
    zpad = jnp.zeros((pe,), jnp.int32)
    npad = jnp.full((pe,), N, jnp.int32)
    inda_g = jnp.concatenate([inda32, zpad])
    indb_g = jnp.concatenate([indb32, zpad])
    inda_s = jnp.concatenate([inda32, npad])
    indb_s = jnp.concatenate([indb32, npad])

    species3 = species.astype(f32).reshape(N // BN, 1, BN)
    inde3 = inde.astype(f32).reshape(N // BN, 1, BN)

    wm4r = [W_m4[l].reshape(64, F, 5).transpose(0, 2, 1).reshape(64, 5 * F)
            for l in range(2)]
    wscr = [W_sc[l].reshape(S, 2 * F) for l in range(2)]

    w_node0 = (W_dn[0, 0], W_dn[0, 1], W_skip[0], wscr[0], W_post[0, 0],
               W_post[0, 1], W_ro0, W_up[1, 0], W_up[1, 1])
    w_e1 = (W_dn[1, 0], wscr[1], W_post[1, 0], W_mlp, W_ro1)

    tx0 = _k_init(species3, emb, W_up[0, 0])
    xs0 = _sc_gather(tx0, inda_g, 32)
    wm4a0 = jnp.concatenate([wm4r[0][:, 0:32], wm4r[0][:, 64:96]], axis=1)
    wm4b1 = wm4r[1][:, 0:64]
    m0 = _k_edge_fwd0(vecs_p, xs0, W_m1[0], W_m2[0], W_m3[0], wm4a0)
    parts0 = _sc_scatter(m0, indb_s, 128)
    p0a, p0b = parts0[0], parts0[1]
    tx1, e0 = _k_node0_fwd(p0a, p0b, species3, w_node0)
    xs1 = _sc_gather(tx1, inda_g, 128)
    m1 = _k_edge_fwd(vecs_p, xs1, W_m1[1], W_m2[1], W_m3[1], wm4r[1])
    parts1 = _sc_scatter(m1, indb_s, 128)
    a1p0, a1p1 = parts1[0, :, 0:32], parts1[1, :, 0:32]
    eg = _k_final_e(a1p0, a1p1, species3, inde3, e0, w_e1)

    dagg1 = _k_node1_bwd(a1p0, a1p1, species3, w_e1)
    dm1 = _sc_gather(dagg1, indb_g, 32)
    dv1, dxs1 = _k_edge_bwd1(vecs_p, xs1, dm1, W_m1[1], W_m2[1], W_m3[1],
                             wm4b1)
    dtx = _sc_scatter(dxs1, inda_s, 128)
    dagg0 = _k_node0_bwd(p0a, p0b, species3, dtx[0], dtx[1], w_node0)
    dm0 = _sc_gather(dagg0, indb_g, 128)
    dv0 = _k_edge_bwd0(vecs_p, xs0, dm0, W_m1[0], W_m2[0], W_m3[0], wm4a0)

    va, vb = _k_fo_vals(dv0, dv1, mask_p)
    vals2 = jnp.concatenate([va, vb], axis=0)
    idx2 = jnp.concatenate([inda_s, indb_s])
    fparts = _sc_scatter(vals2, idx2, 16)
    fo = (fparts[0] + fparts[1])[:N, 0:3]

    return eg[0], fo
